# baseline (speedup 1.0000x reference)
import jax
import jax.numpy as jnp
from jax import lax
from jax.experimental import pallas as pl
from jax.experimental.pallas import tpu as pltpu
from jax.experimental.pallas import tpu_sc as plsc

N = 10000
NP = 10240
IN = 128
H = 256
G = 16
F = 3
E = 320000

CK = 128
CHUNKS = 158
TILE_E = CHUNKS * CK
EP = 16 * TILE_E
DEG_EPT = EP // 32
RB = 256
NB = NP // RB

_mesh = plsc.VectorSubcoreMesh(core_axis_name="c", subcore_axis_name="s")


def _deg_body(dstp_hbm, deg_out, idxbuf, acc, red, out640, sh):
    c = lax.axis_index("c")
    s = lax.axis_index("s")
    wid = s * 2 + c
    pltpu.sync_copy(dstp_hbm.at[pl.ds(wid * DEG_EPT, DEG_EPT)], idxbuf)
    zeros16 = jnp.zeros((16,), jnp.float32)
    ones16 = jnp.ones((16,), jnp.float32)

    def zero_body(i, carry):
        acc[pl.ds(i * 16, 16)] = zeros16
        return carry

    lax.fori_loop(0, NP // 16, zero_body, 0)

    def count_body(i, carry):
        idx16 = idxbuf[pl.ds(i * 16, 16)]
        plsc.addupdate_scatter(acc, [idx16], ones16)
        return carry

    lax.fori_loop(0, DEG_EPT // 16, count_body, 0)

    pltpu.sync_copy(acc, sh.at[s])
    plsc.subcore_barrier()
    cols = NP // 16
    pltpu.sync_copy(sh.at[:, pl.ds(s * cols, cols)], red)

    def red_body(v, carry):
        t = red[0, pl.ds(v * 16, 16)]
        for r in range(1, 16):
            t = t + red[r, pl.ds(v * 16, 16)]
        out640[pl.ds(v * 16, 16)] = t
        return carry

    lax.fori_loop(0, cols // 16, red_body, 0)
    pltpu.sync_copy(out640, deg_out.at[c, pl.ds(s * cols, cols)])


_deg_call = pl.kernel(
    _deg_body,
    out_type=jax.ShapeDtypeStruct((2, NP), jnp.float32),
    mesh=_mesh,
    compiler_params=pltpu.CompilerParams(needs_layout_passes=False),
    scratch_types=[
        pltpu.VMEM((DEG_EPT,), jnp.int32),
        pltpu.VMEM((NP,), jnp.float32),
        pltpu.VMEM((16, NP // 16), jnp.float32),
        pltpu.VMEM((NP // 16,), jnp.float32),
        pltpu.VMEM_SHARED((16, NP), jnp.float32),
    ],
)


def _agg_body(y_hbm, srcp_hbm, dstp_hbm, out_hbm,
              rows0, rows1, sidx0, sidx1, d00, d01, d10, d11,
              acc, sem0, sem1):
    c = lax.axis_index("c")
    s = lax.axis_index("s")
    ebase = s * TILE_E
    rows = (rows0, rows1)
    sidx = (sidx0, sidx1)
    didx = ((d00, d01), (d10, d11))
    sems = (sem0, sem1)
    coff = jnp.full((16,), c * NP, jnp.int32)
    zeros16 = jnp.zeros((16,), jnp.float32)

    def zero_body(r, carry):
        for j in range(CK // 16):
            rows0[r, pl.ds(j * 16, 16)] = zeros16
        return carry

    lax.fori_loop(0, CK, zero_body, 0)
    for q in range(NP // 16 // CK):
        pltpu.sync_copy(rows0, acc.at[pl.ds(s * (NP // 16) + q * CK, CK)])
    plsc.subcore_barrier()

    def load_and_fire(k, b, q):
        pltpu.sync_copy(srcp_hbm.at[pl.ds(ebase + k * CK, CK)], sidx[b])
        pltpu.sync_copy(dstp_hbm.at[pl.ds(ebase + k * CK, CK)], didx[b][q])
        for j in range(CK // 16):
            v = sidx[b][pl.ds(j * 16, 16)]
            sidx[b][pl.ds(j * 16, 16)] = v + coff
        pltpu.make_async_copy(y_hbm.at[sidx[b]], rows[b], sems[b]).start()

    load_and_fire(0, 0, 0)
    load_and_fire(1, 1, 0)

    def pair_step(k0, q, last):
        pltpu.make_async_copy(y_hbm.at[sidx[0]], rows[0], sems[0]).wait()
        pltpu.make_async_copy(rows[0], acc.at[didx[0][q]],
                              sems[0]).start(add=True)
        pltpu.make_async_copy(y_hbm.at[sidx[1]], rows[1], sems[1]).wait()
        pltpu.make_async_copy(rows[1], acc.at[didx[1][q]],
                              sems[1]).start(add=True)
        if not last:
            for b in range(2):
                kn = k0 + 2 + b
                pltpu.sync_copy(srcp_hbm.at[pl.ds(ebase + kn * CK, CK)],
                                sidx[b])
                for j in range(CK // 16):
                    v = sidx[b][pl.ds(j * 16, 16)]
                    sidx[b][pl.ds(j * 16, 16)] = v + coff
                pltpu.sync_copy(dstp_hbm.at[pl.ds(ebase + kn * CK, CK)],
                                didx[b][1 - q])
        pltpu.make_async_copy(rows[0], acc.at[didx[0][q]], sems[0]).wait()
        if not last:
            pltpu.make_async_copy(y_hbm.at[sidx[0]], rows[0], sems[0]).start()
        pltpu.make_async_copy(rows[1], acc.at[didx[1][q]], sems[1]).wait()
        if not last:
            pltpu.make_async_copy(y_hbm.at[sidx[1]], rows[1], sems[1]).start()

    def outer(g2, carry):
        for gpar in range(2):
            g = g2 * 2 + gpar
            pair_step(g * 2, gpar, False)
        return carry

    lax.fori_loop(0, (CHUNKS - 2) // 4, outer, 0)
    pair_step(CHUNKS - 2, 0, True)

    plsc.subcore_barrier()
    rows_per_tile = NP // 16
    pltpu.sync_copy(acc.at[pl.ds(s * rows_per_tile, rows_per_tile)],
                    out_hbm.at[c, pl.ds(s * rows_per_tile, rows_per_tile)])


_agg_call = pl.kernel(
    _agg_body,
    out_type=jax.ShapeDtypeStruct((2, NP, 128), jnp.float32),
    mesh=_mesh,
    scratch_types=[
        pltpu.VMEM((CK, 128), jnp.float32),
        pltpu.VMEM((CK, 128), jnp.float32),
        pltpu.VMEM((CK,), jnp.int32),
        pltpu.VMEM((CK,), jnp.int32),
        pltpu.VMEM((CK,), jnp.int32),
        pltpu.VMEM((CK,), jnp.int32),
        pltpu.VMEM((CK,), jnp.int32),
        pltpu.VMEM((CK,), jnp.int32),
        pltpu.VMEM_SHARED((NP, 128), jnp.float32),
        pltpu.SemaphoreType.DMA,
        pltpu.SemaphoreType.DMA,
    ],
)


def _agg(y_st, srcp, dstp):
    return _agg_call(y_st.reshape(2 * NP, 128), srcp, dstp)


def _dinv_col(deg_ref):
    d = deg_ref[0]
    dsum = d[0:1, :] + d[1:2, :] + 1.0
    return lax.rsqrt(dsum).reshape(RB, 1)


def _full(shape):
    return pl.BlockSpec(shape, lambda *args: tuple(0 for _ in shape))


def _mm1_body(deg_ref, x_ref, w_ref, y_ref):
    dcol = _dinv_col(deg_ref)
    z = jnp.dot(x_ref[...], w_ref[...], preferred_element_type=jnp.float32)
    y = z * dcol
    y_ref[0] = y[:, :128]
    y_ref[1] = y[:, 128:]


def _mm1(xp, W1, deg3):
    return pl.pallas_call(
        _mm1_body,
        grid=(NB,),
        in_specs=[
            pl.BlockSpec((1, 2, RB), lambda i: (i, 0, 0)),
            pl.BlockSpec((RB, IN), lambda i: (i, 0)),
            _full((IN, H)),
        ],
        out_specs=pl.BlockSpec((2, RB, 128), lambda i: (0, i, 0)),
        out_shape=jax.ShapeDtypeStruct((2, NP, 128), jnp.float32),
    )(deg3, xp, W1)


def _mid_body(deg_ref, b_ref, s_ref, y_ref, w_ref, o_ref):
    dcol = _dinv_col(deg_ref)
    scat = jnp.concatenate([s_ref[0], s_ref[1]], axis=1)
    ycat = jnp.concatenate([y_ref[0], y_ref[1]], axis=1)
    h = jnp.maximum((scat + ycat) * dcol + b_ref[...], 0.0)
    z = jnp.dot(h, w_ref[...], preferred_element_type=jnp.float32)
    zs = z * dcol
    o_ref[0] = zs[:, :128]
    o_ref[1] = zs[:, 128:]


def _mid(s_st, y_st, deg3, W, b_prev):
    return pl.pallas_call(
        _mid_body,
        grid=(NB,),
        in_specs=[
            pl.BlockSpec((1, 2, RB), lambda i: (i, 0, 0)),
            _full((H,)),
            pl.BlockSpec((2, RB, 128), lambda i: (0, i, 0)),
            pl.BlockSpec((2, RB, 128), lambda i: (0, i, 0)),
            _full((H, H)),
        ],
        out_specs=pl.BlockSpec((2, RB, 128), lambda i: (0, i, 0)),
        out_shape=jax.ShapeDtypeStruct((2, NP, 128), jnp.float32),
    )(deg3, b_prev, s_st, y_st, W)


def _pool_body(deg_ref, b3_ref, gwt_ref, gb_ref, batch_ref, regw_ref,
               regb_ref, s_ref, y_ref, o_ref, m_scr, ss_scr, sp_scr):
    p = pl.program_id(0)
    i = pl.program_id(1)
    dcol = _dinv_col(deg_ref)
    scat = jnp.concatenate([s_ref[0], s_ref[1]], axis=1)
    ycat = jnp.concatenate([y_ref[0], y_ref[1]], axis=1)
    h3 = (scat + ycat) * dcol + b3_ref[...]
    gate = jnp.sum(h3 * gwt_ref[...], axis=1, keepdims=True) + gb_ref[...]
    bcol = batch_ref[...]
    iota = lax.broadcasted_iota(jnp.int32, (RB, G), 1).astype(jnp.float32)
    oh = bcol == iota

    @pl.when((p == 0) & (i == 0))
    def _():
        m_scr[...] = jnp.full((1, G), -1e30, jnp.float32)

    @pl.when(p == 0)
    def _():
        masked = jnp.where(oh, gate, -1e30)
        m_scr[...] = jnp.maximum(m_scr[...],
                                 jnp.max(masked, axis=0, keepdims=True))

    @pl.when(p == 1)
    def _():
        @pl.when(i == 0)
        def _():
            ss_scr[...] = jnp.zeros((1, G), jnp.float32)
            sp_scr[...] = jnp.zeros((G, H), jnp.float32)

        ohf = oh.astype(jnp.float32)
        mn = jnp.sum(ohf * m_scr[...], axis=1, keepdims=True)
        a = jnp.exp(gate - mn)
        ss_scr[...] = ss_scr[...] + jnp.sum(ohf * a, axis=0, keepdims=True)
        sp_scr[...] = sp_scr[...] + lax.dot_general(
            ohf, h3 * a, (((0,), (0,)), ((), ())),
            preferred_element_type=jnp.float32)

        @pl.when(i == NB - 1)
        def _():
            pooled = sp_scr[...] / (ss_scr[...].reshape(G, 1) + 1e-16)
            o_ref[...] = jnp.tanh(
                jnp.dot(pooled, regw_ref[...],
                        preferred_element_type=jnp.float32) + regb_ref[...])


def _pool(s_st, y_st, deg3, b3, gate_WT, gate_b2, batchp, reg_Wp, reg_bp):
    return pl.pallas_call(
        _pool_body,
        grid=(2, NB),
        in_specs=[
            pl.BlockSpec((1, 2, RB), lambda p, i: (i, 0, 0)),
            _full((H,)),
            _full((1, H)),
            _full((1, 1)),
            pl.BlockSpec((RB, 1), lambda p, i: (i, 0)),
            _full((H, 128)),
            _full((1, 128)),
            pl.BlockSpec((2, RB, 128), lambda p, i: (0, i, 0)),
            pl.BlockSpec((2, RB, 128), lambda p, i: (0, i, 0)),
        ],
        out_specs=pl.BlockSpec((G, 128), lambda p, i: (0, 0)),
        out_shape=jax.ShapeDtypeStruct((G, 128), jnp.float32),
        scratch_shapes=[
            pltpu.VMEM((1, G), jnp.float32),
            pltpu.VMEM((1, G), jnp.float32),
            pltpu.VMEM((G, H), jnp.float32),
        ],
    )(deg3, b3, gate_WT, gate_b2, batchp, reg_Wp, reg_bp, s_st, y_st)


def kernel(x, edge_index, batch, W1, b1, W2, b2, W3, b3,
           gate_W, gate_b, reg_W, reg_b):
    src = edge_index[0]
    dst = edge_index[1]
    pad_e = jnp.full((EP - E,), NP - 1, jnp.int32)
    srcp = jnp.concatenate([src, pad_e])
    dstp = jnp.concatenate([dst, pad_e])
    xp = jnp.pad(x, ((0, NP - N), (0, 0)))
    batchp = jnp.concatenate(
        [batch, jnp.full((NP - N,), G, jnp.int32)]
    ).astype(jnp.float32).reshape(NP, 1)


    degp = _deg_call(dstp)
    deg3 = degp.reshape(2, NB, RB).transpose(1, 0, 2)

    y1 = _mm1(xp, W1, deg3)
    s1 = _agg(y1, srcp, dstp)
    y2 = _mid(s1, y1, deg3, W2, b1)
    s2 = _agg(y2, srcp, dstp)
    y3 = _mid(s2, y2, deg3, W3, b2)
    s3 = _agg(y3, srcp, dstp)

    gate_WT = gate_W.reshape(1, H)
    gate_b2 = gate_b.reshape(1, 1)
    reg_Wp = jnp.pad(reg_W, ((0, 0), (0, 128 - F)))
    reg_bp = jnp.pad(reg_b, ((0, 128 - F),)).reshape(1, 128)
    out128 = _pool(s3, y3, deg3, b3, gate_WT, gate_b2, batchp,
                   reg_Wp, reg_bp)
    return out128[:G, :F]


# speedup vs baseline: 1.2414x; 1.2414x over previous
import jax
import jax.numpy as jnp
from jax import lax
from jax.experimental import pallas as pl
from jax.experimental.pallas import tpu as pltpu
from jax.experimental.pallas import tpu_sc as plsc

N = 10000
NP = 10240
IN = 128
H = 256
G = 16
F = 3
E = 320000

CK = 128
CHUNKS = 158
TILE_E = CHUNKS * CK
EP = 16 * TILE_E
DEG_EPT = EP // 32
RB = 256
NB = NP // RB

_mesh = plsc.VectorSubcoreMesh(core_axis_name="c", subcore_axis_name="s")


def _deg_body(dstp_hbm, deg_out, idxbuf, acc, red, out640, sh):
    c = lax.axis_index("c")
    s = lax.axis_index("s")
    wid = s * 2 + c
    pltpu.sync_copy(dstp_hbm.at[pl.ds(wid * DEG_EPT, DEG_EPT)], idxbuf)
    zeros16 = jnp.zeros((16,), jnp.float32)
    ones16 = jnp.ones((16,), jnp.float32)

    def zero_body(i, carry):
        acc[pl.ds(i * 16, 16)] = zeros16
        return carry

    lax.fori_loop(0, NP // 16, zero_body, 0)

    def count_body(i, carry):
        idx16 = idxbuf[pl.ds(i * 16, 16)]
        plsc.addupdate_scatter(acc, [idx16], ones16)
        return carry

    lax.fori_loop(0, DEG_EPT // 16, count_body, 0)

    pltpu.sync_copy(acc, sh.at[s])
    plsc.subcore_barrier()
    cols = NP // 16
    pltpu.sync_copy(sh.at[:, pl.ds(s * cols, cols)], red)

    def red_body(v, carry):
        t = red[0, pl.ds(v * 16, 16)]
        for r in range(1, 16):
            t = t + red[r, pl.ds(v * 16, 16)]
        out640[pl.ds(v * 16, 16)] = t
        return carry

    lax.fori_loop(0, cols // 16, red_body, 0)
    pltpu.sync_copy(out640, deg_out.at[c, pl.ds(s * cols, cols)])


_deg_call = pl.kernel(
    _deg_body,
    out_type=jax.ShapeDtypeStruct((2, NP), jnp.float32),
    mesh=_mesh,
    compiler_params=pltpu.CompilerParams(needs_layout_passes=False),
    scratch_types=[
        pltpu.VMEM((DEG_EPT,), jnp.int32),
        pltpu.VMEM((NP,), jnp.float32),
        pltpu.VMEM((16, NP // 16), jnp.float32),
        pltpu.VMEM((NP // 16,), jnp.float32),
        pltpu.VMEM_SHARED((16, NP), jnp.float32),
    ],
)


def _agg_body(y_hbm, srcp_hbm, dstp_hbm, out_hbm,
              rows0, rows1, sidx0, sidx1, d00, d01, d10, d11,
              acc, sem0, sem1):
    c = lax.axis_index("c")
    s = lax.axis_index("s")
    ebase = s * TILE_E
    rows = (rows0, rows1)
    sidx = (sidx0, sidx1)
    didx = ((d00, d01), (d10, d11))
    sems = (sem0, sem1)
    coff = jnp.full((16,), c * NP, jnp.int32)
    zeros16 = jnp.zeros((16,), jnp.float32)

    def zero_body(r, carry):
        for j in range(CK // 16):
            rows0[r, pl.ds(j * 16, 16)] = zeros16
        return carry

    lax.fori_loop(0, CK, zero_body, 0)
    for q in range(NP // 16 // CK):
        pltpu.sync_copy(rows0, acc.at[pl.ds(s * (NP // 16) + q * CK, CK)])
    plsc.subcore_barrier()

    def load_and_fire(k, b, q):
        pltpu.sync_copy(srcp_hbm.at[pl.ds(ebase + k * CK, CK)], sidx[b])
        pltpu.sync_copy(dstp_hbm.at[pl.ds(ebase + k * CK, CK)], didx[b][q])
        for j in range(CK // 16):
            v = sidx[b][pl.ds(j * 16, 16)]
            sidx[b][pl.ds(j * 16, 16)] = v + coff
        pltpu.make_async_copy(y_hbm.at[sidx[b]], rows[b], sems[b]).start()

    load_and_fire(0, 0, 0)
    load_and_fire(1, 1, 0)

    def chunk_step(k, b, q, last):
        pltpu.make_async_copy(y_hbm.at[sidx[b]], rows[b], sems[b]).wait()
        pltpu.make_async_copy(rows[b], acc.at[didx[b][q]],
                              sems[b]).start(add=True)
        if not last:
            pltpu.sync_copy(srcp_hbm.at[pl.ds(ebase + (k + 2) * CK, CK)],
                            sidx[b])
            for j in range(CK // 16):
                v = sidx[b][pl.ds(j * 16, 16)]
                sidx[b][pl.ds(j * 16, 16)] = v + coff
            pltpu.sync_copy(dstp_hbm.at[pl.ds(ebase + (k + 2) * CK, CK)],
                            didx[b][1 - q])
        pltpu.make_async_copy(rows[b], acc.at[didx[b][q]], sems[b]).wait()
        if not last:
            pltpu.make_async_copy(y_hbm.at[sidx[b]], rows[b], sems[b]).start()

    def outer(g2, carry):
        for gpar in range(2):
            g = g2 * 2 + gpar
            for b in range(2):
                chunk_step(g * 2 + b, b, gpar, False)
        return carry

    lax.fori_loop(0, (CHUNKS - 2) // 4, outer, 0)
    chunk_step(CHUNKS - 2, 0, 0, True)
    chunk_step(CHUNKS - 1, 1, 0, True)

    plsc.subcore_barrier()
    rows_per_tile = NP // 16
    pltpu.sync_copy(acc.at[pl.ds(s * rows_per_tile, rows_per_tile)],
                    out_hbm.at[c, pl.ds(s * rows_per_tile, rows_per_tile)])


_agg_call = pl.kernel(
    _agg_body,
    out_type=jax.ShapeDtypeStruct((2, NP, 128), jnp.float32),
    mesh=_mesh,
    scratch_types=[
        pltpu.VMEM((CK, 128), jnp.float32),
        pltpu.VMEM((CK, 128), jnp.float32),
        pltpu.VMEM((CK,), jnp.int32),
        pltpu.VMEM((CK,), jnp.int32),
        pltpu.VMEM((CK,), jnp.int32),
        pltpu.VMEM((CK,), jnp.int32),
        pltpu.VMEM((CK,), jnp.int32),
        pltpu.VMEM((CK,), jnp.int32),
        pltpu.VMEM_SHARED((NP, 128), jnp.float32),
        pltpu.SemaphoreType.DMA,
        pltpu.SemaphoreType.DMA,
    ],
)


def _agg(y_st, srcp, dstp):
    return _agg_call(y_st.reshape(2 * NP, 128), srcp, dstp)


def _dinv_col(deg_ref):
    d = deg_ref[0]
    dsum = d[0:1, :] + d[1:2, :] + 1.0
    return lax.rsqrt(dsum).reshape(RB, 1)


def _full(shape):
    return pl.BlockSpec(shape, lambda *args: tuple(0 for _ in shape))


def _mm1_body(deg_ref, x_ref, w_ref, y_ref):
    dcol = _dinv_col(deg_ref)
    z = jnp.dot(x_ref[...], w_ref[...], preferred_element_type=jnp.float32)
    y = z * dcol
    y_ref[0] = y[:, :128]
    y_ref[1] = y[:, 128:]


def _mm1(xp, W1, deg3):
    return pl.pallas_call(
        _mm1_body,
        grid=(NB,),
        in_specs=[
            pl.BlockSpec((1, 2, RB), lambda i: (i, 0, 0)),
            pl.BlockSpec((RB, IN), lambda i: (i, 0)),
            _full((IN, H)),
        ],
        out_specs=pl.BlockSpec((2, RB, 128), lambda i: (0, i, 0)),
        out_shape=jax.ShapeDtypeStruct((2, NP, 128), jnp.float32),
    )(deg3, xp, W1)


def _mid_body(deg_ref, b_ref, s_ref, y_ref, w_ref, o_ref):
    dcol = _dinv_col(deg_ref)
    scat = jnp.concatenate([s_ref[0], s_ref[1]], axis=1)
    ycat = jnp.concatenate([y_ref[0], y_ref[1]], axis=1)
    h = jnp.maximum((scat + ycat) * dcol + b_ref[...], 0.0)
    z = jnp.dot(h, w_ref[...], preferred_element_type=jnp.float32)
    zs = z * dcol
    o_ref[0] = zs[:, :128]
    o_ref[1] = zs[:, 128:]


def _mid(s_st, y_st, deg3, W, b_prev):
    return pl.pallas_call(
        _mid_body,
        grid=(NB,),
        in_specs=[
            pl.BlockSpec((1, 2, RB), lambda i: (i, 0, 0)),
            _full((H,)),
            pl.BlockSpec((2, RB, 128), lambda i: (0, i, 0)),
            pl.BlockSpec((2, RB, 128), lambda i: (0, i, 0)),
            _full((H, H)),
        ],
        out_specs=pl.BlockSpec((2, RB, 128), lambda i: (0, i, 0)),
        out_shape=jax.ShapeDtypeStruct((2, NP, 128), jnp.float32),
    )(deg3, b_prev, s_st, y_st, W)


def _pool_body(deg_ref, b3_ref, gwt_ref, gb_ref, batch_ref, regw_ref,
               regb_ref, s_ref, y_ref, o_ref, m_scr, ss_scr, sp_scr):
    p = pl.program_id(0)
    i = pl.program_id(1)
    dcol = _dinv_col(deg_ref)
    scat = jnp.concatenate([s_ref[0], s_ref[1]], axis=1)
    ycat = jnp.concatenate([y_ref[0], y_ref[1]], axis=1)
    h3 = (scat + ycat) * dcol + b3_ref[...]
    gate = jnp.sum(h3 * gwt_ref[...], axis=1, keepdims=True) + gb_ref[...]
    bcol = batch_ref[...]
    iota = lax.broadcasted_iota(jnp.int32, (RB, G), 1).astype(jnp.float32)
    oh = bcol == iota

    @pl.when((p == 0) & (i == 0))
    def _():
        m_scr[...] = jnp.full((1, G), -1e30, jnp.float32)

    @pl.when(p == 0)
    def _():
        masked = jnp.where(oh, gate, -1e30)
        m_scr[...] = jnp.maximum(m_scr[...],
                                 jnp.max(masked, axis=0, keepdims=True))

    @pl.when(p == 1)
    def _():
        @pl.when(i == 0)
        def _():
            ss_scr[...] = jnp.zeros((1, G), jnp.float32)
            sp_scr[...] = jnp.zeros((G, H), jnp.float32)

        ohf = oh.astype(jnp.float32)
        mn = jnp.sum(ohf * m_scr[...], axis=1, keepdims=True)
        a = jnp.exp(gate - mn)
        ss_scr[...] = ss_scr[...] + jnp.sum(ohf * a, axis=0, keepdims=True)
        sp_scr[...] = sp_scr[...] + lax.dot_general(
            ohf, h3 * a, (((0,), (0,)), ((), ())),
            preferred_element_type=jnp.float32)

        @pl.when(i == NB - 1)
        def _():
            pooled = sp_scr[...] / (ss_scr[...].reshape(G, 1) + 1e-16)
            o_ref[...] = jnp.tanh(
                jnp.dot(pooled, regw_ref[...],
                        preferred_element_type=jnp.float32) + regb_ref[...])


def _pool(s_st, y_st, deg3, b3, gate_WT, gate_b2, batchp, reg_Wp, reg_bp):
    return pl.pallas_call(
        _pool_body,
        grid=(2, NB),
        in_specs=[
            pl.BlockSpec((1, 2, RB), lambda p, i: (i, 0, 0)),
            _full((H,)),
            _full((1, H)),
            _full((1, 1)),
            pl.BlockSpec((RB, 1), lambda p, i: (i, 0)),
            _full((H, 128)),
            _full((1, 128)),
            pl.BlockSpec((2, RB, 128), lambda p, i: (0, i, 0)),
            pl.BlockSpec((2, RB, 128), lambda p, i: (0, i, 0)),
        ],
        out_specs=pl.BlockSpec((G, 128), lambda p, i: (0, 0)),
        out_shape=jax.ShapeDtypeStruct((G, 128), jnp.float32),
        scratch_shapes=[
            pltpu.VMEM((1, G), jnp.float32),
            pltpu.VMEM((1, G), jnp.float32),
            pltpu.VMEM((G, H), jnp.float32),
        ],
    )(deg3, b3, gate_WT, gate_b2, batchp, reg_Wp, reg_bp, s_st, y_st)


def kernel(x, edge_index, batch, W1, b1, W2, b2, W3, b3,
           gate_W, gate_b, reg_W, reg_b):
    src = edge_index[0]
    dst = edge_index[1]
    pad_e = jnp.full((EP - E,), NP - 1, jnp.int32)
    srcp = jnp.concatenate([src, pad_e])
    dstp = jnp.concatenate([dst, pad_e])
    xp = jnp.pad(x, ((0, NP - N), (0, 0)))
    batchp = jnp.concatenate(
        [batch, jnp.full((NP - N,), G, jnp.int32)]
    ).astype(jnp.float32).reshape(NP, 1)


    degp = _deg_call(dstp)
    deg3 = degp.reshape(2, NB, RB).transpose(1, 0, 2)

    y1 = _mm1(xp, W1, deg3)
    s1 = _agg(y1, srcp, dstp)
    y2 = _mid(s1, y1, deg3, W2, b1)
    s2 = _agg(y2, srcp, dstp)
    y3 = _mid(s2, y2, deg3, W3, b2)
    s3 = _agg(y3, srcp, dstp)

    gate_WT = gate_W.reshape(1, H)
    gate_b2 = gate_b.reshape(1, 1)
    reg_Wp = jnp.pad(reg_W, ((0, 0), (0, 128 - F)))
    reg_bp = jnp.pad(reg_b, ((0, 128 - F),)).reshape(1, 128)
    out128 = _pool(s3, y3, deg3, b3, gate_WT, gate_b2, batchp,
                   reg_Wp, reg_bp)
    return out128[:G, :F]
